# baseline (speedup 1.0000x reference)
import jax
import jax.numpy as jnp
from jax import lax
from jax.experimental import pallas as pl
from jax.experimental.pallas import tpu as pltpu
from jax.experimental.pallas import tpu_sc as plsc

WIN_AREA = 1024
NHEADS = 16
NDIST = 3969

NC, NS, L = 2, 16, 16
NW = NC * NS
CHUNK = WIN_AREA
UNROLL = 4

PIECE_ROWS = (64, 320, 320, 320)
PIECE_OFF = tuple(sum(PIECE_ROWS[:k]) for k in range(len(PIECE_ROWS)))


def _sc_bias_kernel(rows_p, row_off, table_hbm, idx_hbm, bias_hbm, table_v,
                    idx_v, buf_a, buf_b, sem_t, sem_i, sem_a, sem_b):
    n_ch_p = rows_p // NW
    pos_p = n_ch_p * CHUNK
    wid = lax.axis_index("s") * NC + lax.axis_index("c")
    row0 = wid * n_ch_p
    base = (row_off + row0) * WIN_AREA

    tcopy = pltpu.async_copy(table_hbm, table_v, sem_t)
    icopy = pltpu.async_copy(idx_hbm.at[pl.ds(base, pos_p)], idx_v, sem_i)
    tcopy.wait()
    icopy.wait()

    def gather_chunk(c, buf):
        @plsc.parallel_loop(0, CHUNK // L, unroll=UNROLL)
        def group_body(g):
            iv = idx_v[pl.ds(c * CHUNK + g * L, L)] * NHEADS
            for h in range(NHEADS):
                buf[h, pl.ds(g * L, L)] = plsc.load_gather(table_v, [iv + h])

    def put_chunk(c, buf, sem):
        pltpu.async_copy(buf, bias_hbm.at[:, row0 + c], sem)

    def wait_chunk(buf, sem):
        pltpu.make_async_copy(buf, bias_hbm.at[:, row0], sem).wait()

    gather_chunk(0, buf_a)
    put_chunk(0, buf_a, sem_a)
    gather_chunk(1, buf_b)
    put_chunk(1, buf_b, sem_b)

    def pair_body(p, _):
        c = p * 2
        wait_chunk(buf_a, sem_a)
        gather_chunk(c, buf_a)
        put_chunk(c, buf_a, sem_a)
        wait_chunk(buf_b, sem_b)
        gather_chunk(c + 1, buf_b)
        put_chunk(c + 1, buf_b, sem_b)
        return ()

    lax.fori_loop(1, n_ch_p // 2, pair_body, (), unroll=False)
    wait_chunk(buf_a, sem_a)
    wait_chunk(buf_b, sem_b)


def _sc_build_bias_piece(k, table, idx):
    rows_p, row_off = PIECE_ROWS[k], PIECE_OFF[k]
    pos_p = rows_p // NW * CHUNK
    mesh = plsc.VectorSubcoreMesh(core_axis_name="c", subcore_axis_name="s")
    return pl.kernel(
        lambda *refs: _sc_bias_kernel(rows_p, row_off, *refs),
        out_type=jax.ShapeDtypeStruct((NHEADS, rows_p, WIN_AREA),
                                      jnp.float32),
        mesh=mesh,
        compiler_params=pltpu.CompilerParams(needs_layout_passes=False),
        scratch_types=[
            pltpu.VMEM((NDIST * NHEADS,), jnp.float32),
            pltpu.VMEM((pos_p,), jnp.int32),
            pltpu.VMEM((NHEADS, CHUNK), jnp.float32),
            pltpu.VMEM((NHEADS, CHUNK), jnp.float32),
            pltpu.SemaphoreType.DMA,
            pltpu.SemaphoreType.DMA,
            pltpu.SemaphoreType.DMA,
            pltpu.SemaphoreType.DMA,
        ],
        name=f"sc_bias_gather_{k}",
    )(table, idx)


BI = 16


def _tc_add_first_kernel(attn_ref, bias_ref, out_ref):
    out_ref[...] = attn_ref[...] + bias_ref[...][None]


def _tc_add_chain_kernel(prev_ref, attn_ref, bias_ref, out_ref):
    del prev_ref
    out_ref[...] = attn_ref[...] + bias_ref[...][None]


def _tc_add_piece(k, prev_out, attn, bias_p):
    nb = attn.shape[0]
    nblk = PIECE_ROWS[k] // BI
    blk0 = PIECE_OFF[k] // BI
    data_spec = pl.BlockSpec((nb, NHEADS, BI, WIN_AREA),
                             lambda ib: (0, 0, blk0 + ib, 0))
    bias_spec = pl.BlockSpec((NHEADS, BI, WIN_AREA), lambda ib: (0, ib, 0))
    out_shape = jax.ShapeDtypeStruct(attn.shape, attn.dtype)
    if k == 0:
        return pl.pallas_call(
            _tc_add_first_kernel,
            grid=(nblk,),
            in_specs=[data_spec, bias_spec],
            out_specs=data_spec,
            out_shape=out_shape,
        )(attn, bias_p)
    prev_spec = pl.BlockSpec((1, 1, 8, 128), lambda ib: (0, 0, 0, 0))
    return pl.pallas_call(
        _tc_add_chain_kernel,
        grid=(nblk,),
        in_specs=[prev_spec, data_spec, bias_spec],
        out_specs=data_spec,
        out_shape=out_shape,
        input_output_aliases={0: 0},
    )(prev_out, attn, bias_p)


@jax.jit
def kernel(attn, relative_position_bias_table, relative_position_index):
    table_flat = relative_position_bias_table.reshape(-1)
    biases = [_sc_build_bias_piece(k, table_flat, relative_position_index)
              for k in range(len(PIECE_ROWS))]
    out = None
    for k in range(len(PIECE_ROWS)):
        out = _tc_add_piece(k, out, attn, biases[k])
    return out


# speedup vs baseline: 1.0274x; 1.0274x over previous
import jax
import jax.numpy as jnp
from jax import lax
from jax.experimental import pallas as pl
from jax.experimental.pallas import tpu as pltpu
from jax.experimental.pallas import tpu_sc as plsc

WIN_AREA = 1024
NHEADS = 16
NDIST = 3969

NC, NS, L = 2, 16, 16
NW = NC * NS
CHUNK = WIN_AREA
UNROLL = 4

PIECE_ROWS = (128, 256, 320, 320)
PIECE_OFF = tuple(sum(PIECE_ROWS[:k]) for k in range(len(PIECE_ROWS)))


def _sc_bias_kernel(rows_p, row_off, table_hbm, idx_hbm, bias_hbm, table_v,
                    idx_v, buf_a, buf_b, sem_t, sem_i, sem_a, sem_b):
    n_ch_p = rows_p // NW
    pos_p = n_ch_p * CHUNK
    wid = lax.axis_index("s") * NC + lax.axis_index("c")
    row0 = wid * n_ch_p
    base = (row_off + row0) * WIN_AREA

    tcopy = pltpu.async_copy(table_hbm, table_v, sem_t)
    icopy = pltpu.async_copy(idx_hbm.at[pl.ds(base, pos_p)], idx_v, sem_i)
    tcopy.wait()
    icopy.wait()

    def gather_chunk(c, buf):
        @plsc.parallel_loop(0, CHUNK // L, unroll=UNROLL)
        def group_body(g):
            iv = idx_v[pl.ds(c * CHUNK + g * L, L)] * NHEADS
            for h in range(NHEADS):
                buf[h, pl.ds(g * L, L)] = plsc.load_gather(table_v, [iv + h])

    def put_chunk(c, buf, sem):
        pltpu.async_copy(buf, bias_hbm.at[:, row0 + c], sem)

    def wait_chunk(buf, sem):
        pltpu.make_async_copy(buf, bias_hbm.at[:, row0], sem).wait()

    gather_chunk(0, buf_a)
    put_chunk(0, buf_a, sem_a)
    gather_chunk(1, buf_b)
    put_chunk(1, buf_b, sem_b)

    def pair_body(p, _):
        c = p * 2
        wait_chunk(buf_a, sem_a)
        gather_chunk(c, buf_a)
        put_chunk(c, buf_a, sem_a)
        wait_chunk(buf_b, sem_b)
        gather_chunk(c + 1, buf_b)
        put_chunk(c + 1, buf_b, sem_b)
        return ()

    lax.fori_loop(1, n_ch_p // 2, pair_body, (), unroll=False)
    wait_chunk(buf_a, sem_a)
    wait_chunk(buf_b, sem_b)


def _sc_build_bias_piece(k, table, idx):
    rows_p, row_off = PIECE_ROWS[k], PIECE_OFF[k]
    pos_p = rows_p // NW * CHUNK
    mesh = plsc.VectorSubcoreMesh(core_axis_name="c", subcore_axis_name="s")
    return pl.kernel(
        lambda *refs: _sc_bias_kernel(rows_p, row_off, *refs),
        out_type=jax.ShapeDtypeStruct((NHEADS, rows_p, WIN_AREA),
                                      jnp.float32),
        mesh=mesh,
        compiler_params=pltpu.CompilerParams(needs_layout_passes=False),
        scratch_types=[
            pltpu.VMEM((NDIST * NHEADS,), jnp.float32),
            pltpu.VMEM((pos_p,), jnp.int32),
            pltpu.VMEM((NHEADS, CHUNK), jnp.float32),
            pltpu.VMEM((NHEADS, CHUNK), jnp.float32),
            pltpu.SemaphoreType.DMA,
            pltpu.SemaphoreType.DMA,
            pltpu.SemaphoreType.DMA,
            pltpu.SemaphoreType.DMA,
        ],
        name=f"sc_bias_gather_{k}",
    )(table, idx)


BI = 16


def _tc_add_first_kernel(attn_ref, bias_ref, out_ref):
    out_ref[...] = attn_ref[...] + bias_ref[...][None]


def _tc_add_chain_kernel(prev_ref, attn_ref, bias_ref, out_ref):
    del prev_ref
    out_ref[...] = attn_ref[...] + bias_ref[...][None]


def _tc_add_piece(k, prev_out, attn, bias_p):
    nb = attn.shape[0]
    nblk = PIECE_ROWS[k] // BI
    blk0 = PIECE_OFF[k] // BI
    data_spec = pl.BlockSpec((nb, NHEADS, BI, WIN_AREA),
                             lambda ib: (0, 0, blk0 + ib, 0))
    bias_spec = pl.BlockSpec((NHEADS, BI, WIN_AREA), lambda ib: (0, ib, 0))
    out_shape = jax.ShapeDtypeStruct(attn.shape, attn.dtype)
    if k == 0:
        return pl.pallas_call(
            _tc_add_first_kernel,
            grid=(nblk,),
            in_specs=[data_spec, bias_spec],
            out_specs=data_spec,
            out_shape=out_shape,
        )(attn, bias_p)
    prev_spec = pl.BlockSpec((1, 1, 8, 128), lambda ib: (0, 0, 0, 0))
    return pl.pallas_call(
        _tc_add_chain_kernel,
        grid=(nblk,),
        in_specs=[prev_spec, data_spec, bias_spec],
        out_specs=data_spec,
        out_shape=out_shape,
        input_output_aliases={0: 0},
    )(prev_out, attn, bias_p)


@jax.jit
def kernel(attn, relative_position_bias_table, relative_position_index):
    table_flat = relative_position_bias_table.reshape(-1)
    biases = [_sc_build_bias_piece(k, table_flat, relative_position_index)
              for k in range(len(PIECE_ROWS))]
    out = None
    for k in range(len(PIECE_ROWS)):
        out = _tc_add_piece(k, out, attn, biases[k])
    return out


# speedup vs baseline: 1.0495x; 1.0215x over previous
import jax
import jax.numpy as jnp
from jax import lax
from jax.experimental import pallas as pl
from jax.experimental.pallas import tpu as pltpu
from jax.experimental.pallas import tpu_sc as plsc

WIN_AREA = 1024
NHEADS = 16
NDIST = 3969

NC, NS, L = 2, 16, 16
NW = NC * NS
CHUNK = WIN_AREA
UNROLL = 4

PIECE_ROWS = (128, 384, 512)
PIECE_OFF = tuple(sum(PIECE_ROWS[:k]) for k in range(len(PIECE_ROWS)))


def _sc_bias_kernel(rows_p, row_off, table_hbm, idx_hbm, bias_hbm, table_v,
                    idx_v, buf_a, buf_b, sem_t, sem_i, sem_a, sem_b):
    n_ch_p = rows_p // NW
    pos_p = n_ch_p * CHUNK
    wid = lax.axis_index("s") * NC + lax.axis_index("c")
    row0 = wid * n_ch_p
    base = (row_off + row0) * WIN_AREA

    tcopy = pltpu.async_copy(table_hbm, table_v, sem_t)
    icopy = pltpu.async_copy(idx_hbm.at[pl.ds(base, pos_p)], idx_v, sem_i)
    tcopy.wait()
    icopy.wait()

    def gather_chunk(c, buf):
        @plsc.parallel_loop(0, CHUNK // L, unroll=UNROLL)
        def group_body(g):
            iv = idx_v[pl.ds(c * CHUNK + g * L, L)] * NHEADS
            for h in range(NHEADS):
                buf[h, pl.ds(g * L, L)] = plsc.load_gather(table_v, [iv + h])

    def put_chunk(c, buf, sem):
        pltpu.async_copy(buf, bias_hbm.at[:, row0 + c], sem)

    def wait_chunk(buf, sem):
        pltpu.make_async_copy(buf, bias_hbm.at[:, row0], sem).wait()

    gather_chunk(0, buf_a)
    put_chunk(0, buf_a, sem_a)
    gather_chunk(1, buf_b)
    put_chunk(1, buf_b, sem_b)

    def pair_body(p, _):
        c = p * 2
        wait_chunk(buf_a, sem_a)
        gather_chunk(c, buf_a)
        put_chunk(c, buf_a, sem_a)
        wait_chunk(buf_b, sem_b)
        gather_chunk(c + 1, buf_b)
        put_chunk(c + 1, buf_b, sem_b)
        return ()

    lax.fori_loop(1, n_ch_p // 2, pair_body, (), unroll=False)
    wait_chunk(buf_a, sem_a)
    wait_chunk(buf_b, sem_b)


def _sc_build_bias_piece(k, table, idx):
    rows_p, row_off = PIECE_ROWS[k], PIECE_OFF[k]
    pos_p = rows_p // NW * CHUNK
    mesh = plsc.VectorSubcoreMesh(core_axis_name="c", subcore_axis_name="s")
    return pl.kernel(
        lambda *refs: _sc_bias_kernel(rows_p, row_off, *refs),
        out_type=jax.ShapeDtypeStruct((NHEADS, rows_p, WIN_AREA),
                                      jnp.float32),
        mesh=mesh,
        compiler_params=pltpu.CompilerParams(needs_layout_passes=False),
        scratch_types=[
            pltpu.VMEM((NDIST * NHEADS,), jnp.float32),
            pltpu.VMEM((pos_p,), jnp.int32),
            pltpu.VMEM((NHEADS, CHUNK), jnp.float32),
            pltpu.VMEM((NHEADS, CHUNK), jnp.float32),
            pltpu.SemaphoreType.DMA,
            pltpu.SemaphoreType.DMA,
            pltpu.SemaphoreType.DMA,
            pltpu.SemaphoreType.DMA,
        ],
        name=f"sc_bias_gather_{k}",
    )(table, idx)


BI = 16


def _tc_add_first_kernel(attn_ref, bias_ref, out_ref):
    out_ref[...] = attn_ref[...] + bias_ref[...][None]


def _tc_add_chain_kernel(prev_ref, attn_ref, bias_ref, out_ref):
    del prev_ref
    out_ref[...] = attn_ref[...] + bias_ref[...][None]


def _tc_add_piece(k, prev_out, attn, bias_p):
    nb = attn.shape[0]
    nblk = PIECE_ROWS[k] // BI
    blk0 = PIECE_OFF[k] // BI
    data_spec = pl.BlockSpec((nb, NHEADS, BI, WIN_AREA),
                             lambda ib: (0, 0, blk0 + ib, 0))
    bias_spec = pl.BlockSpec((NHEADS, BI, WIN_AREA), lambda ib: (0, ib, 0))
    out_shape = jax.ShapeDtypeStruct(attn.shape, attn.dtype)
    if k == 0:
        return pl.pallas_call(
            _tc_add_first_kernel,
            grid=(nblk,),
            in_specs=[data_spec, bias_spec],
            out_specs=data_spec,
            out_shape=out_shape,
        )(attn, bias_p)
    prev_spec = pl.BlockSpec((1, 1, 8, 128), lambda ib: (0, 0, 0, 0))
    return pl.pallas_call(
        _tc_add_chain_kernel,
        grid=(nblk,),
        in_specs=[prev_spec, data_spec, bias_spec],
        out_specs=data_spec,
        out_shape=out_shape,
        input_output_aliases={0: 0},
    )(prev_out, attn, bias_p)


@jax.jit
def kernel(attn, relative_position_bias_table, relative_position_index):
    table_flat = relative_position_bias_table.reshape(-1)
    biases = [_sc_build_bias_piece(k, table_flat, relative_position_index)
              for k in range(len(PIECE_ROWS))]
    out = None
    for k in range(len(PIECE_ROWS)):
        out = _tc_add_piece(k, out, attn, biases[k])
    return out


# speedup vs baseline: 1.0534x; 1.0037x over previous
import jax
import jax.numpy as jnp
from jax import lax
from jax.experimental import pallas as pl
from jax.experimental.pallas import tpu as pltpu
from jax.experimental.pallas import tpu_sc as plsc

WIN_AREA = 1024
NHEADS = 16
NDIST = 3969

NC, NS, L = 2, 16, 16
NW = NC * NS
CHUNK = WIN_AREA
UNROLL = 4

PIECE_ROWS = (128, 384, 512)
PIECE_OFF = tuple(sum(PIECE_ROWS[:k]) for k in range(len(PIECE_ROWS)))


def _sc_bias_kernel(rows_p, row_off, table_hbm, idx_hbm, bias_hbm, table_v,
                    idx_v, buf_a, buf_b, sem_t, sem_i, sem_a, sem_b):
    n_ch_p = rows_p // NW
    pos_p = n_ch_p * CHUNK
    wid = lax.axis_index("s") * NC + lax.axis_index("c")
    row0 = wid * n_ch_p
    base = (row_off + row0) * WIN_AREA

    tcopy = pltpu.async_copy(table_hbm, table_v, sem_t)
    icopy = pltpu.async_copy(idx_hbm.at[pl.ds(base, pos_p)], idx_v, sem_i)
    tcopy.wait()
    icopy.wait()

    def gather_chunk(c, buf):
        @plsc.parallel_loop(0, CHUNK // L, unroll=UNROLL)
        def group_body(g):
            iv = idx_v[pl.ds(c * CHUNK + g * L, L)] * NHEADS
            for h in range(NHEADS):
                buf[h, pl.ds(g * L, L)] = plsc.load_gather(table_v, [iv + h])

    def put_chunk(c, buf, sem):
        pltpu.async_copy(buf, bias_hbm.at[:, row0 + c], sem)

    def wait_chunk(buf, sem):
        pltpu.make_async_copy(buf, bias_hbm.at[:, row0], sem).wait()

    gather_chunk(0, buf_a)
    put_chunk(0, buf_a, sem_a)
    gather_chunk(1, buf_b)
    put_chunk(1, buf_b, sem_b)

    def pair_body(p, _):
        c = p * 2
        wait_chunk(buf_a, sem_a)
        gather_chunk(c, buf_a)
        put_chunk(c, buf_a, sem_a)
        wait_chunk(buf_b, sem_b)
        gather_chunk(c + 1, buf_b)
        put_chunk(c + 1, buf_b, sem_b)
        return ()

    lax.fori_loop(1, n_ch_p // 2, pair_body, (), unroll=False)
    wait_chunk(buf_a, sem_a)
    wait_chunk(buf_b, sem_b)


def _sc_build_bias_piece(k, table, idx):
    rows_p, row_off = PIECE_ROWS[k], PIECE_OFF[k]
    pos_p = rows_p // NW * CHUNK
    mesh = plsc.VectorSubcoreMesh(core_axis_name="c", subcore_axis_name="s")
    return pl.kernel(
        lambda *refs: _sc_bias_kernel(rows_p, row_off, *refs),
        out_type=jax.ShapeDtypeStruct((NHEADS, rows_p, WIN_AREA),
                                      jnp.float32),
        mesh=mesh,
        compiler_params=pltpu.CompilerParams(needs_layout_passes=False),
        scratch_types=[
            pltpu.VMEM((NDIST * NHEADS,), jnp.float32),
            pltpu.VMEM((pos_p,), jnp.int32),
            pltpu.VMEM((NHEADS, CHUNK), jnp.float32),
            pltpu.VMEM((NHEADS, CHUNK), jnp.float32),
            pltpu.SemaphoreType.DMA,
            pltpu.SemaphoreType.DMA,
            pltpu.SemaphoreType.DMA,
            pltpu.SemaphoreType.DMA,
        ],
        name=f"sc_bias_gather_{k}",
    )(table, idx)


BI = 32


def _tc_add_first_kernel(attn_ref, bias_ref, out_ref):
    out_ref[...] = attn_ref[...] + bias_ref[...][None]


def _tc_add_chain_kernel(prev_ref, attn_ref, bias_ref, out_ref):
    del prev_ref
    out_ref[...] = attn_ref[...] + bias_ref[...][None]


HB = 8


def _tc_add_piece(k, prev_out, attn, bias_p):
    nb = attn.shape[0]
    nblk = PIECE_ROWS[k] // BI
    blk0 = PIECE_OFF[k] // BI
    data_spec = pl.BlockSpec((nb, HB, BI, WIN_AREA),
                             lambda ib, ih: (0, ih, blk0 + ib, 0))
    bias_spec = pl.BlockSpec((HB, BI, WIN_AREA),
                             lambda ib, ih: (ih, ib, 0))
    out_shape = jax.ShapeDtypeStruct(attn.shape, attn.dtype)
    grid = (nblk, NHEADS // HB)
    if k == 0:
        return pl.pallas_call(
            _tc_add_first_kernel,
            grid=grid,
            in_specs=[data_spec, bias_spec],
            out_specs=data_spec,
            out_shape=out_shape,
        )(attn, bias_p)
    prev_spec = pl.BlockSpec((1, 1, 8, 128), lambda ib, ih: (0, 0, 0, 0))
    return pl.pallas_call(
        _tc_add_chain_kernel,
        grid=grid,
        in_specs=[prev_spec, data_spec, bias_spec],
        out_specs=data_spec,
        out_shape=out_shape,
        input_output_aliases={0: 0},
    )(prev_out, attn, bias_p)


@jax.jit
def kernel(attn, relative_position_bias_table, relative_position_index):
    table_flat = relative_position_bias_table.reshape(-1)
    biases = [_sc_build_bias_piece(k, table_flat, relative_position_index)
              for k in range(len(PIECE_ROWS))]
    out = None
    for k in range(len(PIECE_ROWS)):
        out = _tc_add_piece(k, out, attn, biases[k])
    return out


# speedup vs baseline: 1.0663x; 1.0123x over previous
import jax
import jax.numpy as jnp
from jax import lax
from jax.experimental import pallas as pl
from jax.experimental.pallas import tpu as pltpu
from jax.experimental.pallas import tpu_sc as plsc

WIN_AREA = 1024
NHEADS = 16
NDIST = 3969

NC, NS, L = 2, 16, 16
NW = NC * NS
CHUNK = WIN_AREA
UNROLL = 4

PIECE_ROWS = (128, 320, 576)
PIECE_OFF = tuple(sum(PIECE_ROWS[:k]) for k in range(len(PIECE_ROWS)))


def _sc_bias_kernel(rows_p, row_off, table_hbm, idx_hbm, bias_hbm, table_v,
                    idx_v, buf_a, buf_b, sem_t, sem_i, sem_a, sem_b):
    n_ch_p = rows_p // NW
    pos_p = n_ch_p * CHUNK
    wid = lax.axis_index("s") * NC + lax.axis_index("c")
    row0 = wid * n_ch_p
    base = (row_off + row0) * WIN_AREA

    tcopy = pltpu.async_copy(table_hbm, table_v, sem_t)
    icopy = pltpu.async_copy(idx_hbm.at[pl.ds(base, pos_p)], idx_v, sem_i)
    tcopy.wait()
    icopy.wait()

    def gather_chunk(c, buf):
        @plsc.parallel_loop(0, CHUNK // L, unroll=UNROLL)
        def group_body(g):
            iv = idx_v[pl.ds(c * CHUNK + g * L, L)] * NHEADS
            for h in range(NHEADS):
                buf[h, pl.ds(g * L, L)] = plsc.load_gather(table_v, [iv + h])

    def put_chunk(c, buf, sem):
        pltpu.async_copy(buf, bias_hbm.at[:, row0 + c], sem)

    def wait_chunk(buf, sem):
        pltpu.make_async_copy(buf, bias_hbm.at[:, row0], sem).wait()

    gather_chunk(0, buf_a)
    put_chunk(0, buf_a, sem_a)
    gather_chunk(1, buf_b)
    put_chunk(1, buf_b, sem_b)

    def pair_body(p, _):
        c = p * 2
        wait_chunk(buf_a, sem_a)
        gather_chunk(c, buf_a)
        put_chunk(c, buf_a, sem_a)
        wait_chunk(buf_b, sem_b)
        gather_chunk(c + 1, buf_b)
        put_chunk(c + 1, buf_b, sem_b)
        return ()

    lax.fori_loop(1, n_ch_p // 2, pair_body, (), unroll=False)
    wait_chunk(buf_a, sem_a)
    wait_chunk(buf_b, sem_b)


def _sc_build_bias_piece(k, table, idx):
    rows_p, row_off = PIECE_ROWS[k], PIECE_OFF[k]
    pos_p = rows_p // NW * CHUNK
    mesh = plsc.VectorSubcoreMesh(core_axis_name="c", subcore_axis_name="s")
    return pl.kernel(
        lambda *refs: _sc_bias_kernel(rows_p, row_off, *refs),
        out_type=jax.ShapeDtypeStruct((NHEADS, rows_p, WIN_AREA),
                                      jnp.float32),
        mesh=mesh,
        compiler_params=pltpu.CompilerParams(needs_layout_passes=False),
        scratch_types=[
            pltpu.VMEM((NDIST * NHEADS,), jnp.float32),
            pltpu.VMEM((pos_p,), jnp.int32),
            pltpu.VMEM((NHEADS, CHUNK), jnp.float32),
            pltpu.VMEM((NHEADS, CHUNK), jnp.float32),
            pltpu.SemaphoreType.DMA,
            pltpu.SemaphoreType.DMA,
            pltpu.SemaphoreType.DMA,
            pltpu.SemaphoreType.DMA,
        ],
        name=f"sc_bias_gather_{k}",
    )(table, idx)


BI = 32


def _tc_add_first_kernel(attn_ref, bias_ref, out_ref):
    out_ref[...] = attn_ref[...] + bias_ref[...][None]


def _tc_add_chain_kernel(prev_ref, attn_ref, bias_ref, out_ref):
    del prev_ref
    out_ref[...] = attn_ref[...] + bias_ref[...][None]


HB = 8


def _tc_add_piece(k, prev_out, attn, bias_p):
    nb = attn.shape[0]
    nblk = PIECE_ROWS[k] // BI
    blk0 = PIECE_OFF[k] // BI
    data_spec = pl.BlockSpec((nb, HB, BI, WIN_AREA),
                             lambda ib, ih: (0, ih, blk0 + ib, 0))
    bias_spec = pl.BlockSpec((HB, BI, WIN_AREA),
                             lambda ib, ih: (ih, ib, 0))
    out_shape = jax.ShapeDtypeStruct(attn.shape, attn.dtype)
    grid = (nblk, NHEADS // HB)
    if k == 0:
        return pl.pallas_call(
            _tc_add_first_kernel,
            grid=grid,
            in_specs=[data_spec, bias_spec],
            out_specs=data_spec,
            out_shape=out_shape,
        )(attn, bias_p)
    prev_spec = pl.BlockSpec((1, 1, 8, 128), lambda ib, ih: (0, 0, 0, 0))
    return pl.pallas_call(
        _tc_add_chain_kernel,
        grid=grid,
        in_specs=[prev_spec, data_spec, bias_spec],
        out_specs=data_spec,
        out_shape=out_shape,
        input_output_aliases={0: 0},
    )(prev_out, attn, bias_p)


@jax.jit
def kernel(attn, relative_position_bias_table, relative_position_index):
    table_flat = relative_position_bias_table.reshape(-1)
    biases = [_sc_build_bias_piece(k, table_flat, relative_position_index)
              for k in range(len(PIECE_ROWS))]
    out = None
    for k in range(len(PIECE_ROWS)):
        out = _tc_add_piece(k, out, attn, biases[k])
    return out
